# baseline (speedup 1.0000x reference)
import functools
import math

import jax
import jax.numpy as jnp
from jax import lax
from jax.experimental import pallas as pl
from jax.experimental.pallas import tpu as pltpu
from jax.experimental.pallas import tpu_sc as plsc

NC = 2
NS = 16
NW = NC * NS
LANES = 16
NBUF = 8
G1 = 128


def _emb_kernel(B0, T, V, D):
    rows_per_w = B0 // NW
    scale = math.sqrt(D)
    g2 = T - G1
    mesh = plsc.VectorSubcoreMesh(
        core_axis_name="c", subcore_axis_name="s", num_cores=NC, num_subcores=NS
    )
    assert rows_per_w % NBUF == 0 and rows_per_w > NBUF + 4

    @functools.partial(
        pl.kernel,
        mesh=mesh,
        out_type=jax.ShapeDtypeStruct((B0, T, D), jnp.float32),
        compiler_params=pltpu.CompilerParams(use_tc_tiling_on_sc=False),
        scratch_types=[
            pltpu.VMEM((NBUF, T), jnp.int32),
            pltpu.VMEM((NBUF, T, D), jnp.float32),
        ]
        + [pltpu.SemaphoreType.DMA] * (3 * NBUF),
    )
    def k(tok_hbm, table_hbm, out_hbm, idx_v, rows_v, *sems):
        isem = sems[:NBUF]
        gsem = sems[NBUF : 2 * NBUF]
        wsem = sems[2 * NBUF :]
        wid = lax.axis_index("s") * NC + lax.axis_index("c")
        base = wid * rows_per_w

        def fire_idx(j, b):
            pltpu.async_copy(tok_hbm.at[base + j], idx_v.at[b], isem[b])

        def wait_idx(b):
            pltpu.make_async_copy(tok_hbm.at[0], idx_v.at[b], isem[b]).wait()

        def fire_gathers(b):
            pltpu.async_copy(
                table_hbm.at[idx_v.at[b, pl.ds(0, G1)]],
                rows_v.at[b, pl.ds(0, G1)],
                gsem[b],
            )
            pltpu.async_copy(
                table_hbm.at[idx_v.at[b, pl.ds(G1, g2)]],
                rows_v.at[b, pl.ds(G1, g2)],
                gsem[b],
            )

        def wait_gathers(b):
            pltpu.make_async_copy(out_hbm.at[0], rows_v.at[b], gsem[b]).wait()

        def fire_write(j, b):
            pltpu.async_copy(rows_v.at[b], out_hbm.at[base + j], wsem[b])

        def wait_write(b):
            pltpu.make_async_copy(out_hbm.at[0], rows_v.at[b], wsem[b]).wait()

        def do_scale(b):
            def scale_body(i, c):
                for r in range(2):
                    for t in range(D // LANES):
                        sl = pl.ds(t * LANES, LANES)
                        rows_v[b, 2 * i + r, sl] = rows_v[b, 2 * i + r, sl] * scale
                return c

            lax.fori_loop(0, T // 2, scale_body, 0)

        def process(j, b):
            wait_gathers(b)
            do_scale(b)
            fire_write(j, b)

        for j in range(4):
            fire_idx(j, j)
        for j in range(2):
            wait_idx(j)
            fire_gathers(j)

        for j in range(4):
            process(j, j)
            fire_idx(j + 4, (j + 4) % NBUF)
            wait_idx((j + 2) % NBUF)
            fire_gathers((j + 2) % NBUF)

        def body(m, carry):
            j0 = 4 + m * NBUF
            for u in range(NBUF):
                j = j0 + u
                b = (4 + u) % NBUF
                process(j, b)
                b4 = (4 + u + 4) % NBUF
                wait_write(b4)
                fire_idx(j + 4, b4)
                b2 = (4 + u + 2) % NBUF
                wait_idx(b2)
                fire_gathers(b2)
            return carry

        lax.fori_loop(0, (rows_per_w - 8) // NBUF, body, 0)

        for u in range(4):
            j = rows_per_w - 4 + u
            process(j, j % NBUF)
            if u < 2:
                wait_idx((j + 2) % NBUF)
                fire_gathers((j + 2) % NBUF)

        for b in range(NBUF):
            wait_write(b)

    return k


def kernel(tokens, table):
    B0, T = tokens.shape
    V, D = table.shape
    assert B0 % NW == 0 and D % LANES == 0 and T % 2 == 0 and G1 < T <= G1 + 128
    return _emb_kernel(B0, T, V, D)(tokens.astype(jnp.int32), table)
